# Initial kernel scaffold; baseline (speedup 1.0000x reference)
import jax
import jax.numpy as jnp
from jax.experimental import pallas as pl


def kernel(input, embedding_weight):
    raise NotImplementedError("write your pallas kernel here")


# speedup vs baseline: 1.6839x; 1.6839x over previous
import functools

import jax
import jax.numpy as jnp
from jax import lax
from jax.experimental import pallas as pl
from jax.experimental.pallas import tpu as pltpu
from jax.experimental.pallas import tpu_sc as plsc

_NC = 2
_NS = 16
_NW = _NC * _NS
_C = 128


@functools.lru_cache(maxsize=None)
def _gather_call(n_rows, d):
    chunks_total = n_rows // _C
    chunks_per_w = chunks_total // _NW
    mesh = plsc.VectorSubcoreMesh(core_axis_name="c", subcore_axis_name="s")

    @functools.partial(
        pl.kernel,
        mesh=mesh,
        out_type=jax.ShapeDtypeStruct((n_rows, d), jnp.float32),
        compiler_params=pltpu.CompilerParams(use_tc_tiling_on_sc=False),
        scratch_types=[
            pltpu.VMEM((chunks_per_w, _C), jnp.int32),
            pltpu.VMEM((_C, d), jnp.float32),
            pltpu.SemaphoreType.DMA,
        ],
    )
    def k(idx_hbm, table_hbm, out_hbm, idx_v, rows_v, sem):
        wid = lax.axis_index("s") * _NC + lax.axis_index("c")
        c0 = wid * chunks_per_w
        pltpu.sync_copy(idx_hbm.at[pl.ds(c0, chunks_per_w)], idx_v)

        def body(j, carry):
            pltpu.async_copy(table_hbm.at[idx_v.at[j]], rows_v, sem).wait()
            pltpu.sync_copy(rows_v, out_hbm.at[pl.ds((c0 + j) * _C, _C)])
            return carry

        lax.fori_loop(0, chunks_per_w, body, 0)

    return k


def kernel(input, embedding_weight):
    b, h = input.shape
    v, d = embedding_weight.shape
    n = b * h
    idx2d = input.reshape(n // _C, _C)
    out = _gather_call(n, d)(idx2d, embedding_weight)
    return out.reshape(b, h, d)


# speedup vs baseline: 1.8709x; 1.1111x over previous
import functools

import jax
import jax.numpy as jnp
from jax import lax
from jax.experimental import pallas as pl
from jax.experimental.pallas import tpu as pltpu
from jax.experimental.pallas import tpu_sc as plsc

_NC = 2
_NS = 16
_NW = _NC * _NS
_C = 128
_K = 4


@functools.lru_cache(maxsize=None)
def _gather_call(n_rows, d):
    chunks_total = n_rows // _C
    chunks_per_w = chunks_total // _NW
    groups = chunks_per_w // _K
    pairs = groups // 2
    rows_per_group = _K * _C
    mesh = plsc.VectorSubcoreMesh(core_axis_name="c", subcore_axis_name="s")

    @functools.partial(
        pl.kernel,
        mesh=mesh,
        out_type=jax.ShapeDtypeStruct((n_rows, d), jnp.float32),
        compiler_params=pltpu.CompilerParams(use_tc_tiling_on_sc=False),
        scratch_types=[
            pltpu.VMEM((chunks_per_w, _C), jnp.int32),
            pltpu.VMEM((rows_per_group, d), jnp.float32),
            pltpu.VMEM((rows_per_group, d), jnp.float32),
            pltpu.SemaphoreType.DMA,
            pltpu.SemaphoreType.DMA,
            pltpu.SemaphoreType.DMA,
            pltpu.SemaphoreType.DMA,
        ],
    )
    def k(idx_hbm, table_hbm, out_hbm, idx_v, rows0, rows1,
          gsem0, gsem1, ssem0, ssem1):
        wid = lax.axis_index("s") * _NC + lax.axis_index("c")
        c0 = wid * chunks_per_w
        pltpu.sync_copy(idx_hbm.at[pl.ds(c0, chunks_per_w)], idx_v)
        bufs = (rows0, rows1)
        gsems = (gsem0, gsem1)
        ssems = (ssem0, ssem1)

        def out_slice(g):
            return out_hbm.at[pl.ds((c0 + g * _K) * _C, rows_per_group)]

        def store_desc(b, g):
            return pltpu.make_async_copy(bufs[b], out_slice(g), ssems[b])

        def gather_drain_desc(b, g):
            return pltpu.make_async_copy(out_slice(g), bufs[b], gsems[b])

        def do_group(b, g, first):
            if not first:
                store_desc(b, g).wait()
            for kk in range(_K):
                pltpu.async_copy(
                    table_hbm.at[idx_v.at[g * _K + kk]],
                    bufs[b].at[pl.ds(kk * _C, _C)],
                    gsems[b])
            gather_drain_desc(b, g).wait()
            store_desc(b, g).start()

        do_group(0, 0, True)
        do_group(1, 1, True)

        def body(u, carry):
            for b in range(2):
                do_group(b, u * 2 + b, False)
            return carry

        lax.fori_loop(1, pairs, body, 0)

        store_desc(0, (pairs - 1) * 2).wait()
        store_desc(1, (pairs - 1) * 2 + 1).wait()

    return k


def kernel(input, embedding_weight):
    b, h = input.shape
    v, d = embedding_weight.shape
    n = b * h
    idx2d = input.reshape(n // _C, _C)
    out = _gather_call(n, d)(idx2d, embedding_weight)
    return out.reshape(b, h, d)
